# Initial kernel scaffold; baseline (speedup 1.0000x reference)
import functools

import jax
import jax.numpy as jnp
from jax import lax
from jax.experimental import pallas as pl
from jax.experimental.pallas import tpu as pltpu
from jax.experimental.pallas import tpu_sc as plsc

NC = 2
NS = 16
NW = NC * NS
CHUNK = 128


def _sc_mesh():
    return plsc.VectorSubcoreMesh(core_axis_name="c", subcore_axis_name="s")


def _sc_degree(dst_rs, ones_blk, zeros, n_pad, cpw):
    rpt = n_pad // NS

    @functools.partial(
        pl.kernel,
        out_type=jax.ShapeDtypeStruct((NC, n_pad, 16), jnp.float32),
        mesh=_sc_mesh(),
        scratch_types=[
            pltpu.VMEM((cpw, CHUNK), jnp.int32),
            pltpu.VMEM((CHUNK, 16), jnp.float32),
            pltpu.VMEM_SHARED((n_pad, 16), jnp.float32),
        ],
    )
    def k(dst_hbm, ones_hbm, zeros_hbm, out_hbm, idx_v, ones_v, acc):
        c = lax.axis_index("c")
        s = lax.axis_index("s")
        w = c * NS + s
        r0 = s * rpt
        pltpu.sync_copy(zeros_hbm.at[pl.ds(r0, rpt)], acc.at[pl.ds(r0, rpt)])
        pltpu.sync_copy(dst_hbm.at[pl.ds(w * cpw, cpw)], idx_v)
        pltpu.sync_copy(ones_hbm, ones_v)
        plsc.subcore_barrier()

        @pl.loop(0, cpw)
        def _(i):
            pltpu.sync_copy(ones_v, acc.at[idx_v.at[i]], add=True)

        plsc.subcore_barrier()
        pltpu.sync_copy(acc.at[pl.ds(r0, rpt)], out_hbm.at[c, pl.ds(r0, rpt)])

    return k(dst_rs, ones_blk, zeros)


def _sc_aggregate(h, src_rs, dst_rs, zeros, n_pad, cpw, d):
    rpt = n_pad // NS

    @functools.partial(
        pl.kernel,
        out_type=jax.ShapeDtypeStruct((NC, n_pad, d), jnp.float32),
        mesh=_sc_mesh(),
        scratch_types=[
            pltpu.VMEM((cpw, CHUNK), jnp.int32),
            pltpu.VMEM((cpw, CHUNK), jnp.int32),
            pltpu.VMEM((CHUNK, d), jnp.float32),
            pltpu.VMEM_SHARED((n_pad, d), jnp.float32),
            pltpu.SemaphoreType.DMA,
        ],
    )
    def k(h_hbm, src_hbm, dst_hbm, zeros_hbm, out_hbm,
          src_v, dst_v, rows_v, acc, sem):
        c = lax.axis_index("c")
        s = lax.axis_index("s")
        w = c * NS + s
        r0 = s * rpt
        pltpu.sync_copy(zeros_hbm.at[pl.ds(r0, rpt)], acc.at[pl.ds(r0, rpt)])
        pltpu.sync_copy(src_hbm.at[pl.ds(w * cpw, cpw)], src_v)
        pltpu.sync_copy(dst_hbm.at[pl.ds(w * cpw, cpw)], dst_v)
        plsc.subcore_barrier()

        @pl.loop(0, cpw)
        def _(i):
            pltpu.async_copy(h_hbm.at[src_v.at[i]], rows_v, sem).wait()
            pltpu.sync_copy(rows_v, acc.at[dst_v.at[i]], add=True)

        plsc.subcore_barrier()
        pltpu.sync_copy(acc.at[pl.ds(r0, rpt)], out_hbm.at[c, pl.ds(r0, rpt)])

    return k(h, src_rs, dst_rs, zeros)


def _dot(a, b):
    return jnp.dot(a, b, preferred_element_type=jnp.float32,
                   precision=lax.Precision.HIGHEST)


def _tc(body, out_shape, *args):
    return pl.pallas_call(body, out_shape=out_shape)(*args)


def kernel(x, edge_index, W1, b1, W2, b2, W3, b3, W4, b4):
    n, d_in = x.shape
    e = edge_index.shape[1]
    c4 = W4.shape[1]
    c4p = ((c4 + 15) // 16) * 16

    n_pad = (n // NS + 1) * NS
    cpw = -(-e // (NW * CHUNK))
    e_pad = cpw * NW * CHUNK

    src = edge_index[0].astype(jnp.int32)
    dst = edge_index[1].astype(jnp.int32)
    pad = e_pad - e
    src_rs = jnp.concatenate([src, jnp.zeros((pad,), jnp.int32)])
    dst_rs = jnp.concatenate([dst, jnp.full((pad,), n, jnp.int32)])
    src_rs = src_rs.reshape(e_pad // CHUNK, CHUNK)
    dst_rs = dst_rs.reshape(e_pad // CHUNK, CHUNK)

    ones_blk = jnp.ones((CHUNK, 16), jnp.float32)
    z16 = jnp.zeros((n_pad, 16), jnp.float32)
    z64 = jnp.zeros((n_pad, 64), jnp.float32)
    zc4 = jnp.zeros((n_pad, c4p), jnp.float32)

    b1r = b1.reshape(1, -1)
    b2r = b2.reshape(1, -1)
    b3r = b3.reshape(1, -1)
    b4r = b4.reshape(1, -1)
    W4p = jnp.pad(W4, ((0, 0), (0, c4p - c4)))

    dp = _sc_degree(dst_rs, ones_blk, z16, n_pad, cpw)

    def k1(x_ref, w_ref, dp0_ref, dp1_ref, h1p_ref, p_ref):
        deg = dp0_ref[:, 0:1] + dp1_ref[:, 0:1] + 1.0
        p = lax.rsqrt(deg)[:n]
        h1p_ref[...] = p * _dot(x_ref[...], w_ref[...])
        p_ref[...] = p

    h1p, p = _tc(k1, (jax.ShapeDtypeStruct((n, W1.shape[1]), jnp.float32),
                      jax.ShapeDtypeStruct((n, 1), jnp.float32)),
                 x, W1, dp[0], dp[1])

    g1 = _sc_aggregate(h1p, src_rs, dst_rs, z64, n_pad, cpw, W1.shape[1])

    def k2(g0_ref, g1_ref, h1p_ref, p_ref, b_ref, t2_ref):
        z1 = jnp.maximum(
            p_ref[...] * (g0_ref[:n, :] + g1_ref[:n, :] + h1p_ref[...])
            + b_ref[...], 0.0)
        t2_ref[...] = p_ref[...] * z1

    t2 = _tc(k2, jax.ShapeDtypeStruct((n, W1.shape[1]), jnp.float32),
             g1[0], g1[1], h1p, p, b1r)

    g2 = _sc_aggregate(t2, src_rs, dst_rs, z64, n_pad, cpw, W1.shape[1])

    def k3(g0_ref, g1_ref, t2_ref, p_ref, w2_ref, b2_ref, w3_ref, h3p_ref):
        a2 = p_ref[...] * (g0_ref[:n, :] + g1_ref[:n, :] + t2_ref[...])
        z2 = jnp.maximum(_dot(a2, w2_ref[...]) + b2_ref[...], 0.0)
        h3p_ref[...] = p_ref[...] * _dot(z2, w3_ref[...])

    h3p = _tc(k3, jax.ShapeDtypeStruct((n, W3.shape[1]), jnp.float32),
              g2[0], g2[1], t2, p, W2, b2r, W3)

    g3 = _sc_aggregate(h3p, src_rs, dst_rs, z64, n_pad, cpw, W3.shape[1])

    def k4(g0_ref, g1_ref, h3p_ref, p_ref, b3_ref, w4_ref, h4p_ref):
        z3 = jnp.maximum(
            p_ref[...] * (g0_ref[:n, :] + g1_ref[:n, :] + h3p_ref[...])
            + b3_ref[...], 0.0)
        h4p_ref[...] = p_ref[...] * _dot(z3, w4_ref[...])

    h4p = _tc(k4, jax.ShapeDtypeStruct((n, c4p), jnp.float32),
              g3[0], g3[1], h3p, p, b3r, W4p)

    g4 = _sc_aggregate(h4p, src_rs, dst_rs, zc4, n_pad, cpw, c4p)

    def k5(g0_ref, g1_ref, h4p_ref, p_ref, b4_ref, out_ref):
        z4 = p_ref[...] * (g0_ref[:n, :] + g1_ref[:n, :] + h4p_ref[...])
        z = z4[:, :c4] + b4_ref[...]
        m = jnp.max(z, axis=1, keepdims=True)
        zs = z - m
        lse = jnp.log(jnp.sum(jnp.exp(zs), axis=1, keepdims=True))
        out_ref[...] = zs - lse

    return _tc(k5, jax.ShapeDtypeStruct((n, c4), jnp.float32),
               g4[0], g4[1], h4p, p, b4r)


# speedup vs baseline: 9.5728x; 9.5728x over previous
import functools

import jax
import jax.numpy as jnp
from jax import lax
from jax.experimental import pallas as pl
from jax.experimental.pallas import tpu as pltpu
from jax.experimental.pallas import tpu_sc as plsc

NC = 2
NS = 16
NW = NC * NS
CHUNK = 128


def _sc_mesh():
    return plsc.VectorSubcoreMesh(core_axis_name="c", subcore_axis_name="s")


_SC_PARAMS = pltpu.CompilerParams(use_tc_tiling_on_sc=False)


def _sc_degree(dst_rs, ones_blk, zeros, n_pad, cpw):
    rpt = n_pad // NS

    @functools.partial(
        pl.kernel,
        out_type=jax.ShapeDtypeStruct((NC, n_pad, 16), jnp.float32),
        mesh=_sc_mesh(),
        compiler_params=_SC_PARAMS,
        scratch_types=[
            pltpu.VMEM((cpw, CHUNK), jnp.int32),
            pltpu.VMEM((CHUNK, 16), jnp.float32),
            pltpu.VMEM_SHARED((n_pad, 16), jnp.float32),
        ],
    )
    def k(dst_hbm, ones_hbm, zeros_hbm, out_hbm, idx_v, ones_v, acc):
        c = lax.axis_index("c")
        s = lax.axis_index("s")
        w = c * NS + s
        r0 = s * rpt
        pltpu.sync_copy(zeros_hbm.at[pl.ds(r0, rpt)], acc.at[pl.ds(r0, rpt)])
        pltpu.sync_copy(dst_hbm.at[pl.ds(w * cpw, cpw)], idx_v)
        pltpu.sync_copy(ones_hbm, ones_v)
        plsc.subcore_barrier()

        @pl.loop(0, cpw)
        def _(i):
            pltpu.sync_copy(ones_v, acc.at[idx_v.at[i]], add=True)

        plsc.subcore_barrier()
        pltpu.sync_copy(acc.at[pl.ds(r0, rpt)], out_hbm.at[c, pl.ds(r0, rpt)])

    return k(dst_rs, ones_blk, zeros)


def _sc_aggregate(h, src_rs, dst_rs, zeros, n_pad, cpw, d):
    rpt = n_pad // NS

    @functools.partial(
        pl.kernel,
        out_type=jax.ShapeDtypeStruct((NC, n_pad, d), jnp.float32),
        mesh=_sc_mesh(),
        compiler_params=_SC_PARAMS,
        scratch_types=[
            pltpu.VMEM((cpw, CHUNK), jnp.int32),
            pltpu.VMEM((cpw, CHUNK), jnp.int32),
            pltpu.VMEM((CHUNK, d), jnp.float32),
            pltpu.VMEM_SHARED((n_pad, d), jnp.float32),
            pltpu.SemaphoreType.DMA,
        ],
    )
    def k(h_hbm, src_hbm, dst_hbm, zeros_hbm, out_hbm,
          src_v, dst_v, rows_v, acc, sem):
        c = lax.axis_index("c")
        s = lax.axis_index("s")
        w = c * NS + s
        r0 = s * rpt
        pltpu.sync_copy(zeros_hbm.at[pl.ds(r0, rpt)], acc.at[pl.ds(r0, rpt)])
        pltpu.sync_copy(src_hbm.at[pl.ds(w * cpw, cpw)], src_v)
        pltpu.sync_copy(dst_hbm.at[pl.ds(w * cpw, cpw)], dst_v)
        plsc.subcore_barrier()

        @pl.loop(0, cpw)
        def _(i):
            pltpu.async_copy(h_hbm.at[src_v.at[i]], rows_v, sem).wait()
            pltpu.sync_copy(rows_v, acc.at[dst_v.at[i]], add=True)

        plsc.subcore_barrier()
        pltpu.sync_copy(acc.at[pl.ds(r0, rpt)], out_hbm.at[c, pl.ds(r0, rpt)])

    return k(h, src_rs, dst_rs, zeros)


def _dot(a, b):
    return jnp.dot(a, b, preferred_element_type=jnp.float32,
                   precision=lax.Precision.HIGHEST)


def _tc(body, out_shape, *args):
    return pl.pallas_call(body, out_shape=out_shape)(*args)


def kernel(x, edge_index, W1, b1, W2, b2, W3, b3, W4, b4):
    n, d_in = x.shape
    e = edge_index.shape[1]
    c4 = W4.shape[1]
    c4p = ((c4 + 15) // 16) * 16

    n_pad = (n // (8 * NS) + 1) * (8 * NS)
    cpw = 8 * -(-e // (NW * CHUNK * 8))
    e_pad = cpw * NW * CHUNK

    src = edge_index[0].astype(jnp.int32)
    dst = edge_index[1].astype(jnp.int32)
    pad = e_pad - e
    src_rs = jnp.concatenate([src, jnp.zeros((pad,), jnp.int32)])
    dst_rs = jnp.concatenate([dst, jnp.full((pad,), n, jnp.int32)])
    src_rs = src_rs.reshape(e_pad // CHUNK, CHUNK)
    dst_rs = dst_rs.reshape(e_pad // CHUNK, CHUNK)

    ones_blk = jnp.ones((CHUNK, 16), jnp.float32)
    z16 = jnp.zeros((n_pad, 16), jnp.float32)
    z64 = jnp.zeros((n_pad, 64), jnp.float32)
    zc4 = jnp.zeros((n_pad, c4p), jnp.float32)

    b1r = b1.reshape(1, -1)
    b2r = b2.reshape(1, -1)
    b3r = b3.reshape(1, -1)
    b4r = b4.reshape(1, -1)
    W4p = jnp.pad(W4, ((0, 0), (0, c4p - c4)))

    dp = _sc_degree(dst_rs, ones_blk, z16, n_pad, cpw)

    def k1(x_ref, w_ref, dp0_ref, dp1_ref, h1p_ref, p_ref):
        deg = dp0_ref[:, 0:1] + dp1_ref[:, 0:1] + 1.0
        p = lax.rsqrt(deg)[:n]
        h1p_ref[...] = p * _dot(x_ref[...], w_ref[...])
        p_ref[...] = p

    h1p, p = _tc(k1, (jax.ShapeDtypeStruct((n, W1.shape[1]), jnp.float32),
                      jax.ShapeDtypeStruct((n, 1), jnp.float32)),
                 x, W1, dp[0], dp[1])

    g1 = _sc_aggregate(h1p, src_rs, dst_rs, z64, n_pad, cpw, W1.shape[1])

    def k2(g0_ref, g1_ref, h1p_ref, p_ref, b_ref, t2_ref):
        z1 = jnp.maximum(
            p_ref[...] * (g0_ref[:n, :] + g1_ref[:n, :] + h1p_ref[...])
            + b_ref[...], 0.0)
        t2_ref[...] = p_ref[...] * z1

    t2 = _tc(k2, jax.ShapeDtypeStruct((n, W1.shape[1]), jnp.float32),
             g1[0], g1[1], h1p, p, b1r)

    g2 = _sc_aggregate(t2, src_rs, dst_rs, z64, n_pad, cpw, W1.shape[1])

    def k3(g0_ref, g1_ref, t2_ref, p_ref, w2_ref, b2_ref, w3_ref, h3p_ref):
        a2 = p_ref[...] * (g0_ref[:n, :] + g1_ref[:n, :] + t2_ref[...])
        z2 = jnp.maximum(_dot(a2, w2_ref[...]) + b2_ref[...], 0.0)
        h3p_ref[...] = p_ref[...] * _dot(z2, w3_ref[...])

    h3p = _tc(k3, jax.ShapeDtypeStruct((n, W3.shape[1]), jnp.float32),
              g2[0], g2[1], t2, p, W2, b2r, W3)

    g3 = _sc_aggregate(h3p, src_rs, dst_rs, z64, n_pad, cpw, W3.shape[1])

    def k4(g0_ref, g1_ref, h3p_ref, p_ref, b3_ref, w4_ref, h4p_ref):
        z3 = jnp.maximum(
            p_ref[...] * (g0_ref[:n, :] + g1_ref[:n, :] + h3p_ref[...])
            + b3_ref[...], 0.0)
        h4p_ref[...] = p_ref[...] * _dot(z3, w4_ref[...])

    h4p = _tc(k4, jax.ShapeDtypeStruct((n, c4p), jnp.float32),
              g3[0], g3[1], h3p, p, b3r, W4p)

    g4 = _sc_aggregate(h4p, src_rs, dst_rs, zc4, n_pad, cpw, c4p)

    def k5(g0_ref, g1_ref, h4p_ref, p_ref, b4_ref, out_ref):
        z4 = p_ref[...] * (g0_ref[:n, :] + g1_ref[:n, :] + h4p_ref[...])
        z = z4[:, :c4] + b4_ref[...]
        m = jnp.max(z, axis=1, keepdims=True)
        zs = z - m
        lse = jnp.log(jnp.sum(jnp.exp(zs), axis=1, keepdims=True))
        out_ref[...] = zs - lse

    return _tc(k5, jax.ShapeDtypeStruct((n, c4), jnp.float32),
               g4[0], g4[1], h4p, p, b4r)


# speedup vs baseline: 26.0817x; 2.7246x over previous
import functools

import jax
import jax.numpy as jnp
from jax import lax
from jax.experimental import pallas as pl
from jax.experimental.pallas import tpu as pltpu
from jax.experimental.pallas import tpu_sc as plsc

NC = 2
NS = 16
NW = NC * NS
CHUNK = 128


def _sc_mesh():
    return plsc.VectorSubcoreMesh(core_axis_name="c", subcore_axis_name="s")


_SC_PARAMS = pltpu.CompilerParams(use_tc_tiling_on_sc=False)


def _sc_degree(dst_rs, ones_blk, zeros, n_pad, cpw):
    rpt = n_pad // NS

    @functools.partial(
        pl.kernel,
        out_type=jax.ShapeDtypeStruct((NC, n_pad, 16), jnp.float32),
        mesh=_sc_mesh(),
        compiler_params=_SC_PARAMS,
        scratch_types=[
            pltpu.VMEM((cpw, CHUNK), jnp.int32),
            pltpu.VMEM((CHUNK, 16), jnp.float32),
            pltpu.VMEM_SHARED((n_pad, 16), jnp.float32),
        ],
    )
    def k(dst_hbm, ones_hbm, zeros_hbm, out_hbm, idx_v, ones_v, acc):
        c = lax.axis_index("c")
        s = lax.axis_index("s")
        w = c * NS + s
        r0 = s * rpt
        pltpu.sync_copy(zeros_hbm.at[pl.ds(r0, rpt)], acc.at[pl.ds(r0, rpt)])
        pltpu.sync_copy(dst_hbm.at[pl.ds(w * cpw, cpw)], idx_v)
        pltpu.sync_copy(ones_hbm, ones_v)
        plsc.subcore_barrier()

        @pl.loop(0, cpw)
        def _(i):
            pltpu.sync_copy(ones_v, acc.at[idx_v.at[i]], add=True)

        plsc.subcore_barrier()
        pltpu.sync_copy(acc.at[pl.ds(r0, rpt)], out_hbm.at[c, pl.ds(r0, rpt)])

    return k(dst_rs, ones_blk, zeros)


def _sc_aggregate(h, src_rs, dst_rs, zeros, n_pad, cpw, d):
    rpt = n_pad // NS
    n = h.shape[0]
    stg = n // NS // 8 * 8
    stg_last = n - stg * (NS - 1)

    @functools.partial(
        pl.kernel,
        out_type=jax.ShapeDtypeStruct((NC, n_pad, d), jnp.float32),
        mesh=_sc_mesh(),
        compiler_params=_SC_PARAMS,
        scratch_types=[
            pltpu.VMEM((cpw + 8, CHUNK), jnp.int32),
            pltpu.VMEM((cpw, CHUNK), jnp.int32),
            pltpu.VMEM((2, CHUNK, d), jnp.float32),
            pltpu.VMEM_SHARED((n, d), jnp.float32),
            pltpu.VMEM_SHARED((n_pad, d), jnp.float32),
            pltpu.SemaphoreType.DMA,
            pltpu.SemaphoreType.DMA,
        ],
    )
    def k(h_hbm, src_hbm, dst_hbm, zeros_hbm, out_hbm,
          src_v, dst_v, rows_v, table, acc, sem0, sem1):
        c = lax.axis_index("c")
        s = lax.axis_index("s")
        w = c * NS + s
        r0 = s * rpt
        @pl.when(s < NS - 1)
        def _():
            pltpu.sync_copy(h_hbm.at[pl.ds(s * stg, stg)],
                            table.at[pl.ds(s * stg, stg)])

        @pl.when(s == NS - 1)
        def _():
            pltpu.sync_copy(h_hbm.at[pl.ds((NS - 1) * stg, stg_last)],
                            table.at[pl.ds((NS - 1) * stg, stg_last)])

        pltpu.sync_copy(zeros_hbm.at[pl.ds(r0, rpt)], acc.at[pl.ds(r0, rpt)])
        pltpu.sync_copy(src_hbm.at[pl.ds(w * cpw, cpw)],
                        src_v.at[pl.ds(0, cpw)])
        pltpu.sync_copy(src_hbm.at[pl.ds(w * cpw, 8)],
                        src_v.at[pl.ds(cpw, 8)])
        pltpu.sync_copy(dst_hbm.at[pl.ds(w * cpw, cpw)], dst_v)
        plsc.subcore_barrier()

        sems = (sem0, sem1)
        pltpu.async_copy(table.at[src_v.at[0]], rows_v.at[0], sem0)
        pltpu.async_copy(table.at[src_v.at[1]], rows_v.at[1], sem1)

        @pl.loop(0, cpw // 2)
        def _(j):
            for b in range(2):
                i = 2 * j + b
                pltpu.make_async_copy(table.at[src_v.at[i]],
                                      rows_v.at[b], sems[b]).wait()
                pltpu.sync_copy(rows_v.at[b], acc.at[dst_v.at[i]], add=True)
                pltpu.async_copy(table.at[src_v.at[i + 2]],
                                 rows_v.at[b], sems[b])

        pltpu.make_async_copy(table.at[src_v.at[cpw]],
                              rows_v.at[0], sem0).wait()
        pltpu.make_async_copy(table.at[src_v.at[cpw + 1]],
                              rows_v.at[1], sem1).wait()
        plsc.subcore_barrier()
        pltpu.sync_copy(acc.at[pl.ds(r0, rpt)], out_hbm.at[c, pl.ds(r0, rpt)])

    return k(h, src_rs, dst_rs, zeros)


def _dot(a, b):
    return jnp.dot(a, b, preferred_element_type=jnp.float32,
                   precision=lax.Precision.HIGHEST)


def _tc(body, out_shape, *args):
    return pl.pallas_call(body, out_shape=out_shape)(*args)


def kernel(x, edge_index, W1, b1, W2, b2, W3, b3, W4, b4):
    n, d_in = x.shape
    e = edge_index.shape[1]
    c4 = W4.shape[1]
    c4p = ((c4 + 15) // 16) * 16

    n_pad = (n // (8 * NS) + 1) * (8 * NS)
    cpw = 8 * -(-e // (NW * CHUNK * 8))
    e_pad = cpw * NW * CHUNK

    src = edge_index[0].astype(jnp.int32)
    dst = edge_index[1].astype(jnp.int32)
    pad = e_pad - e
    src_rs = jnp.concatenate([src, jnp.zeros((pad,), jnp.int32)])
    dst_rs = jnp.concatenate([dst, jnp.full((pad,), n, jnp.int32)])
    src_rs = src_rs.reshape(e_pad // CHUNK, CHUNK)
    dst_rs = dst_rs.reshape(e_pad // CHUNK, CHUNK)

    ones_blk = jnp.ones((CHUNK, 16), jnp.float32)
    z16 = jnp.zeros((n_pad, 16), jnp.float32)
    z64 = jnp.zeros((n_pad, 64), jnp.float32)
    zc4 = jnp.zeros((n_pad, c4p), jnp.float32)

    b1r = b1.reshape(1, -1)
    b2r = b2.reshape(1, -1)
    b3r = b3.reshape(1, -1)
    b4r = b4.reshape(1, -1)
    W4p = jnp.pad(W4, ((0, 0), (0, c4p - c4)))

    dp = _sc_degree(dst_rs, ones_blk, z16, n_pad, cpw)

    def k1(x_ref, w_ref, dp0_ref, dp1_ref, h1p_ref, p_ref):
        deg = dp0_ref[:, 0:1] + dp1_ref[:, 0:1] + 1.0
        p = lax.rsqrt(deg)[:n]
        h1p_ref[...] = p * _dot(x_ref[...], w_ref[...])
        p_ref[...] = p

    h1p, p = _tc(k1, (jax.ShapeDtypeStruct((n, W1.shape[1]), jnp.float32),
                      jax.ShapeDtypeStruct((n, 1), jnp.float32)),
                 x, W1, dp[0], dp[1])

    g1 = _sc_aggregate(h1p, src_rs, dst_rs, z64, n_pad, cpw, W1.shape[1])

    def k2(g0_ref, g1_ref, h1p_ref, p_ref, b_ref, t2_ref):
        z1 = jnp.maximum(
            p_ref[...] * (g0_ref[:n, :] + g1_ref[:n, :] + h1p_ref[...])
            + b_ref[...], 0.0)
        t2_ref[...] = p_ref[...] * z1

    t2 = _tc(k2, jax.ShapeDtypeStruct((n, W1.shape[1]), jnp.float32),
             g1[0], g1[1], h1p, p, b1r)

    g2 = _sc_aggregate(t2, src_rs, dst_rs, z64, n_pad, cpw, W1.shape[1])

    def k3(g0_ref, g1_ref, t2_ref, p_ref, w2_ref, b2_ref, w3_ref, h3p_ref):
        a2 = p_ref[...] * (g0_ref[:n, :] + g1_ref[:n, :] + t2_ref[...])
        z2 = jnp.maximum(_dot(a2, w2_ref[...]) + b2_ref[...], 0.0)
        h3p_ref[...] = p_ref[...] * _dot(z2, w3_ref[...])

    h3p = _tc(k3, jax.ShapeDtypeStruct((n, W3.shape[1]), jnp.float32),
              g2[0], g2[1], t2, p, W2, b2r, W3)

    g3 = _sc_aggregate(h3p, src_rs, dst_rs, z64, n_pad, cpw, W3.shape[1])

    def k4(g0_ref, g1_ref, h3p_ref, p_ref, b3_ref, w4_ref, h4p_ref):
        z3 = jnp.maximum(
            p_ref[...] * (g0_ref[:n, :] + g1_ref[:n, :] + h3p_ref[...])
            + b3_ref[...], 0.0)
        h4p_ref[...] = p_ref[...] * _dot(z3, w4_ref[...])

    h4p = _tc(k4, jax.ShapeDtypeStruct((n, c4p), jnp.float32),
              g3[0], g3[1], h3p, p, b3r, W4p)

    g4 = _sc_aggregate(h4p, src_rs, dst_rs, zc4, n_pad, cpw, c4p)

    def k5(g0_ref, g1_ref, h4p_ref, p_ref, b4_ref, out_ref):
        z4 = p_ref[...] * (g0_ref[:n, :] + g1_ref[:n, :] + h4p_ref[...])
        z = z4[:, :c4] + b4_ref[...]
        m = jnp.max(z, axis=1, keepdims=True)
        zs = z - m
        lse = jnp.log(jnp.sum(jnp.exp(zs), axis=1, keepdims=True))
        out_ref[...] = zs - lse

    return _tc(k5, jax.ShapeDtypeStruct((n, c4), jnp.float32),
               g4[0], g4[1], h4p, p, b4r)


# speedup vs baseline: 26.0918x; 1.0004x over previous
import functools

import jax
import jax.numpy as jnp
from jax import lax
from jax.experimental import pallas as pl
from jax.experimental.pallas import tpu as pltpu
from jax.experimental.pallas import tpu_sc as plsc

NC = 2
NS = 16
NW = NC * NS
CHUNK = 128


def _sc_mesh():
    return plsc.VectorSubcoreMesh(core_axis_name="c", subcore_axis_name="s")


_SC_PARAMS = pltpu.CompilerParams(use_tc_tiling_on_sc=False)


def _sc_degree(dst_rs, ones_blk, zeros, n_pad, cpw):
    rpt = n_pad // NS

    @functools.partial(
        pl.kernel,
        out_type=jax.ShapeDtypeStruct((NC, n_pad, 16), jnp.float32),
        mesh=_sc_mesh(),
        compiler_params=_SC_PARAMS,
        scratch_types=[
            pltpu.VMEM((cpw, CHUNK), jnp.int32),
            pltpu.VMEM((CHUNK, 16), jnp.float32),
            pltpu.VMEM_SHARED((n_pad, 16), jnp.float32),
        ],
    )
    def k(dst_hbm, ones_hbm, zeros_hbm, out_hbm, idx_v, ones_v, acc):
        c = lax.axis_index("c")
        s = lax.axis_index("s")
        w = c * NS + s
        r0 = s * rpt
        pltpu.sync_copy(zeros_hbm.at[pl.ds(r0, rpt)], acc.at[pl.ds(r0, rpt)])
        pltpu.sync_copy(dst_hbm.at[pl.ds(w * cpw, cpw)], idx_v)
        pltpu.sync_copy(ones_hbm, ones_v)
        plsc.subcore_barrier()

        @pl.loop(0, cpw)
        def _(i):
            pltpu.sync_copy(ones_v, acc.at[idx_v.at[i]], add=True)

        plsc.subcore_barrier()
        pltpu.sync_copy(acc.at[pl.ds(r0, rpt)], out_hbm.at[c, pl.ds(r0, rpt)])

    return k(dst_rs, ones_blk, zeros)


def _sc_aggregate(h, src_rs, dst_rs, zeros, n_pad, cpw, d):
    rpt = n_pad // NS
    n = h.shape[0]
    stg = n // NS // 8 * 8
    stg_last = n - stg * (NS - 1)

    @functools.partial(
        pl.kernel,
        out_type=jax.ShapeDtypeStruct((NC, n_pad, d), jnp.float32),
        mesh=_sc_mesh(),
        compiler_params=_SC_PARAMS,
        scratch_types=[
            pltpu.VMEM((cpw + 8, CHUNK), jnp.int32),
            pltpu.VMEM((cpw, CHUNK), jnp.int32),
            pltpu.VMEM((3, CHUNK, d), jnp.float32),
            pltpu.VMEM_SHARED((n, d), jnp.float32),
            pltpu.VMEM_SHARED((n_pad, d), jnp.float32),
            [pltpu.SemaphoreType.DMA] * 4,
        ],
    )
    def k(h_hbm, src_hbm, dst_hbm, zeros_hbm, out_hbm,
          src_v, dst_v, rows_v, table, acc, semg):
        c = lax.axis_index("c")
        s = lax.axis_index("s")
        w = c * NS + s
        r0 = s * rpt
        @pl.when(s < NS - 1)
        def _():
            pltpu.sync_copy(h_hbm.at[pl.ds(s * stg, stg)],
                            table.at[pl.ds(s * stg, stg)])

        @pl.when(s == NS - 1)
        def _():
            pltpu.sync_copy(h_hbm.at[pl.ds((NS - 1) * stg, stg_last)],
                            table.at[pl.ds((NS - 1) * stg, stg_last)])

        pltpu.sync_copy(zeros_hbm.at[pl.ds(r0, rpt)], acc.at[pl.ds(r0, rpt)])
        pltpu.sync_copy(src_hbm.at[pl.ds(w * cpw, cpw)],
                        src_v.at[pl.ds(0, cpw)])
        pltpu.sync_copy(src_hbm.at[pl.ds(w * cpw, 8)],
                        src_v.at[pl.ds(cpw, 8)])
        pltpu.sync_copy(dst_hbm.at[pl.ds(w * cpw, cpw)], dst_v)
        plsc.subcore_barrier()

        def fire_gather(i, b):
            pltpu.async_copy(table.at[src_v.at[i]], rows_v.at[b], semg[b])

        def wait_gather(i, b):
            pltpu.make_async_copy(table.at[src_v.at[i]], rows_v.at[b],
                                  semg[b]).wait()

        for b in range(3):
            fire_gather(b, b)

        nmain = cpw // 3 * 3

        @pl.loop(0, cpw // 3)
        def _(j):
            for k in range(3):
                i = 3 * j + k
                wait_gather(i, k)
                pltpu.sync_copy(rows_v.at[k], acc.at[dst_v.at[i]], add=True)
                fire_gather(i + 3, k)

        for i in range(nmain, cpw):
            wait_gather(i, i % 3)
            pltpu.sync_copy(rows_v.at[i % 3], acc.at[dst_v.at[i]], add=True)
        for i in range(cpw, nmain + 3):
            wait_gather(i, i % 3)
        plsc.subcore_barrier()
        pltpu.sync_copy(acc.at[pl.ds(r0, rpt)], out_hbm.at[c, pl.ds(r0, rpt)])

    return k(h, src_rs, dst_rs, zeros)


def _dot(a, b):
    return jnp.dot(a, b, preferred_element_type=jnp.float32,
                   precision=lax.Precision.HIGHEST)


def _tc(body, out_shape, *args):
    return pl.pallas_call(body, out_shape=out_shape)(*args)


def kernel(x, edge_index, W1, b1, W2, b2, W3, b3, W4, b4):
    n, d_in = x.shape
    e = edge_index.shape[1]
    c4 = W4.shape[1]
    c4p = ((c4 + 15) // 16) * 16

    n_pad = (n // (8 * NS) + 1) * (8 * NS)
    cpw = 8 * -(-e // (NW * CHUNK * 8))
    e_pad = cpw * NW * CHUNK

    src = edge_index[0].astype(jnp.int32)
    dst = edge_index[1].astype(jnp.int32)
    pad = e_pad - e
    src_rs = jnp.concatenate([src, jnp.zeros((pad,), jnp.int32)])
    dst_rs = jnp.concatenate([dst, jnp.full((pad,), n, jnp.int32)])
    src_rs = src_rs.reshape(e_pad // CHUNK, CHUNK)
    dst_rs = dst_rs.reshape(e_pad // CHUNK, CHUNK)

    ones_blk = jnp.ones((CHUNK, 16), jnp.float32)
    z16 = jnp.zeros((n_pad, 16), jnp.float32)
    z64 = jnp.zeros((n_pad, 64), jnp.float32)
    zc4 = jnp.zeros((n_pad, c4p), jnp.float32)

    b1r = b1.reshape(1, -1)
    b2r = b2.reshape(1, -1)
    b3r = b3.reshape(1, -1)
    b4r = b4.reshape(1, -1)
    W4p = jnp.pad(W4, ((0, 0), (0, c4p - c4)))

    dp = _sc_degree(dst_rs, ones_blk, z16, n_pad, cpw)

    def k0(x_ref, w_ref, h1_ref):
        h1_ref[...] = _dot(x_ref[...], w_ref[...])

    h1 = _tc(k0, jax.ShapeDtypeStruct((n, W1.shape[1]), jnp.float32), x, W1)

    def k1(h1_ref, dp0_ref, dp1_ref, h1p_ref, p_ref):
        deg = dp0_ref[:, 0:1] + dp1_ref[:, 0:1] + 1.0
        p = lax.rsqrt(deg)[:n]
        h1p_ref[...] = p * h1_ref[...]
        p_ref[...] = p

    h1p, p = _tc(k1, (jax.ShapeDtypeStruct((n, W1.shape[1]), jnp.float32),
                      jax.ShapeDtypeStruct((n, 1), jnp.float32)),
                 h1, dp[0], dp[1])

    g1 = _sc_aggregate(h1p, src_rs, dst_rs, z64, n_pad, cpw, W1.shape[1])

    def k2(g0_ref, g1_ref, h1p_ref, p_ref, b_ref, t2_ref):
        z1 = jnp.maximum(
            p_ref[...] * (g0_ref[:n, :] + g1_ref[:n, :] + h1p_ref[...])
            + b_ref[...], 0.0)
        t2_ref[...] = p_ref[...] * z1

    t2 = _tc(k2, jax.ShapeDtypeStruct((n, W1.shape[1]), jnp.float32),
             g1[0], g1[1], h1p, p, b1r)

    g2 = _sc_aggregate(t2, src_rs, dst_rs, z64, n_pad, cpw, W1.shape[1])

    def k3(g0_ref, g1_ref, t2_ref, p_ref, w2_ref, b2_ref, w3_ref, h3p_ref):
        a2 = p_ref[...] * (g0_ref[:n, :] + g1_ref[:n, :] + t2_ref[...])
        z2 = jnp.maximum(_dot(a2, w2_ref[...]) + b2_ref[...], 0.0)
        h3p_ref[...] = p_ref[...] * _dot(z2, w3_ref[...])

    h3p = _tc(k3, jax.ShapeDtypeStruct((n, W3.shape[1]), jnp.float32),
              g2[0], g2[1], t2, p, W2, b2r, W3)

    g3 = _sc_aggregate(h3p, src_rs, dst_rs, z64, n_pad, cpw, W3.shape[1])

    def k4(g0_ref, g1_ref, h3p_ref, p_ref, b3_ref, w4_ref, h4p_ref):
        z3 = jnp.maximum(
            p_ref[...] * (g0_ref[:n, :] + g1_ref[:n, :] + h3p_ref[...])
            + b3_ref[...], 0.0)
        h4p_ref[...] = p_ref[...] * _dot(z3, w4_ref[...])

    h4p = _tc(k4, jax.ShapeDtypeStruct((n, c4p), jnp.float32),
              g3[0], g3[1], h3p, p, b3r, W4p)

    g4 = _sc_aggregate(h4p, src_rs, dst_rs, zc4, n_pad, cpw, c4p)

    def k5(g0_ref, g1_ref, h4p_ref, p_ref, b4_ref, out_ref):
        z4 = p_ref[...] * (g0_ref[:n, :] + g1_ref[:n, :] + h4p_ref[...])
        z = z4[:, :c4] + b4_ref[...]
        m = jnp.max(z, axis=1, keepdims=True)
        zs = z - m
        lse = jnp.log(jnp.sum(jnp.exp(zs), axis=1, keepdims=True))
        out_ref[...] = zs - lse

    return _tc(k5, jax.ShapeDtypeStruct((n, c4), jnp.float32),
               g4[0], g4[1], h4p, p, b4r)


# speedup vs baseline: 27.9052x; 1.0695x over previous
import functools

import jax
import jax.numpy as jnp
from jax import lax
from jax.experimental import pallas as pl
from jax.experimental.pallas import tpu as pltpu
from jax.experimental.pallas import tpu_sc as plsc

NC = 2
NS = 16
NW = NC * NS
CHUNK = 128


def _sc_mesh():
    return plsc.VectorSubcoreMesh(core_axis_name="c", subcore_axis_name="s")


_SC_PARAMS = pltpu.CompilerParams(use_tc_tiling_on_sc=False)


def _sc_degree(ei_rs, ones_blk, zeros, n_pad, nchunks):
    rpt = n_pad // NS
    base = nchunks // NW
    extra = nchunks - base * NW

    @functools.partial(
        pl.kernel,
        out_type=jax.ShapeDtypeStruct((NC, n_pad, 16), jnp.float32),
        mesh=_sc_mesh(),
        compiler_params=_SC_PARAMS,
        scratch_types=[
            pltpu.VMEM((base + 1, CHUNK), jnp.int32),
            pltpu.VMEM((CHUNK, 16), jnp.float32),
            pltpu.VMEM_SHARED((n_pad, 16), jnp.float32),
        ],
    )
    def k(ei_hbm, ones_hbm, zeros_hbm, out_hbm, idx_v, ones_v, acc):
        c = lax.axis_index("c")
        s = lax.axis_index("s")
        w = c * NS + s
        r0 = s * rpt
        pltpu.sync_copy(zeros_hbm.at[pl.ds(r0, rpt)], acc.at[pl.ds(r0, rpt)])
        pltpu.sync_copy(ei_hbm.at[1, pl.ds(w * base, base)],
                        idx_v.at[pl.ds(0, base)])

        @pl.when(w < extra)
        def _():
            pltpu.sync_copy(ei_hbm.at[1, pl.ds(base * NW + w, 1)],
                            idx_v.at[pl.ds(base, 1)])

        pltpu.sync_copy(ones_hbm, ones_v)
        plsc.subcore_barrier()

        @pl.loop(0, base)
        def _(i):
            pltpu.sync_copy(ones_v, acc.at[idx_v.at[i]], add=True)

        @pl.when(w < extra)
        def _():
            pltpu.sync_copy(ones_v, acc.at[idx_v.at[base]], add=True)

        plsc.subcore_barrier()
        pltpu.sync_copy(acc.at[pl.ds(r0, rpt)], out_hbm.at[c, pl.ds(r0, rpt)])

    return k(ei_rs, ones_blk, zeros)


def _sc_aggregate(h, ei_rs, zeros, n_pad, nchunks, d):
    rpt = n_pad // NS
    n = h.shape[0]
    stg = n // NS // 8 * 8
    stg_last = n - stg * (NS - 1)
    base = nchunks // NW
    extra = nchunks - base * NW

    @functools.partial(
        pl.kernel,
        out_type=jax.ShapeDtypeStruct((NC, n_pad, d), jnp.float32),
        mesh=_sc_mesh(),
        compiler_params=_SC_PARAMS,
        scratch_types=[
            pltpu.VMEM((base + 8, CHUNK), jnp.int32),
            pltpu.VMEM((base + 1, CHUNK), jnp.int32),
            pltpu.VMEM((3, CHUNK, d), jnp.float32),
            pltpu.VMEM_SHARED((n, d), jnp.float32),
            pltpu.VMEM_SHARED((n_pad, d), jnp.float32),
            [pltpu.SemaphoreType.DMA] * 3,
        ],
    )
    def k(h_hbm, ei_hbm, zeros_hbm, out_hbm,
          src_v, dst_v, rows_v, table, acc, semg):
        c = lax.axis_index("c")
        s = lax.axis_index("s")
        w = c * NS + s
        r0 = s * rpt
        @pl.when(s < NS - 1)
        def _():
            pltpu.sync_copy(h_hbm.at[pl.ds(s * stg, stg)],
                            table.at[pl.ds(s * stg, stg)])

        @pl.when(s == NS - 1)
        def _():
            pltpu.sync_copy(h_hbm.at[pl.ds((NS - 1) * stg, stg_last)],
                            table.at[pl.ds((NS - 1) * stg, stg_last)])

        pltpu.sync_copy(zeros_hbm.at[pl.ds(r0, rpt)], acc.at[pl.ds(r0, rpt)])
        pltpu.sync_copy(ei_hbm.at[0, pl.ds(w * base, base)],
                        src_v.at[pl.ds(0, base)])
        pltpu.sync_copy(ei_hbm.at[0, pl.ds(w * base, 8)],
                        src_v.at[pl.ds(base, 8)])
        pltpu.sync_copy(ei_hbm.at[1, pl.ds(w * base, base)],
                        dst_v.at[pl.ds(0, base)])

        @pl.when(w < extra)
        def _():
            pltpu.sync_copy(ei_hbm.at[0, pl.ds(base * NW + w, 1)],
                            src_v.at[pl.ds(base + 4, 1)])
            pltpu.sync_copy(ei_hbm.at[1, pl.ds(base * NW + w, 1)],
                            dst_v.at[pl.ds(base, 1)])

        plsc.subcore_barrier()

        @pl.when(w < extra)
        def _():
            pltpu.sync_copy(table.at[src_v.at[base + 4]], rows_v.at[0])
            pltpu.sync_copy(rows_v.at[0], acc.at[dst_v.at[base]], add=True)

        def fire_gather(i, b):
            pltpu.async_copy(table.at[src_v.at[i]], rows_v.at[b], semg[b])

        def wait_gather(i, b):
            pltpu.make_async_copy(table.at[src_v.at[i]], rows_v.at[b],
                                  semg[b]).wait()

        for b in range(3):
            fire_gather(b, b)

        nmain = base // 3 * 3

        @pl.loop(0, base // 3)
        def _(j):
            for k in range(3):
                i = 3 * j + k
                wait_gather(i, k)
                pltpu.sync_copy(rows_v.at[k], acc.at[dst_v.at[i]], add=True)
                fire_gather(i + 3, k)

        for i in range(nmain, base):
            wait_gather(i, i % 3)
            pltpu.sync_copy(rows_v.at[i % 3], acc.at[dst_v.at[i]], add=True)
        for i in range(base, nmain + 3):
            wait_gather(i, i % 3)
        plsc.subcore_barrier()
        pltpu.sync_copy(acc.at[pl.ds(r0, rpt)], out_hbm.at[c, pl.ds(r0, rpt)])

    return k(h, ei_rs, zeros)


def _dot(a, b):
    return jnp.dot(a, b, preferred_element_type=jnp.float32,
                   precision=lax.Precision.DEFAULT)


def _tc(body, out_shape, *args):
    return pl.pallas_call(body, out_shape=out_shape)(*args)


def kernel(x, edge_index, W1, b1, W2, b2, W3, b3, W4, b4):
    n, d_in = x.shape
    e = edge_index.shape[1]
    c4 = W4.shape[1]
    c4p = ((c4 + 15) // 16) * 16

    n_pad = -(-n // (8 * NS)) * (8 * NS)
    assert e % CHUNK == 0
    nchunks = e // CHUNK

    ei_rs = edge_index.astype(jnp.int32).reshape(2, nchunks, CHUNK)

    ones_blk = jnp.ones((CHUNK, 16), jnp.float32)
    z16 = jnp.zeros((n_pad, 16), jnp.float32)
    z64 = jnp.zeros((n_pad, 64), jnp.float32)
    zc4 = jnp.zeros((n_pad, c4p), jnp.float32)

    b1r = b1.reshape(1, -1)
    b2r = b2.reshape(1, -1)
    b3r = b3.reshape(1, -1)
    b4r = b4.reshape(1, -1)
    W4p = jnp.pad(W4, ((0, 0), (0, c4p - c4)))

    dp = _sc_degree(ei_rs, ones_blk, z16, n_pad, nchunks)

    def k0(x_ref, w_ref, h1_ref):
        h1_ref[...] = _dot(x_ref[...], w_ref[...])

    h1 = _tc(k0, jax.ShapeDtypeStruct((n, W1.shape[1]), jnp.float32), x, W1)

    def k1(h1_ref, dp0_ref, dp1_ref, h1p_ref, p_ref):
        deg = dp0_ref[:, 0:1] + dp1_ref[:, 0:1] + 1.0
        p = lax.rsqrt(deg)[:n]
        h1p_ref[...] = p * h1_ref[...]
        p_ref[...] = p

    h1p, p = _tc(k1, (jax.ShapeDtypeStruct((n, W1.shape[1]), jnp.float32),
                      jax.ShapeDtypeStruct((n, 1), jnp.float32)),
                 h1, dp[0], dp[1])

    g1 = _sc_aggregate(h1p, ei_rs, z64, n_pad, nchunks, W1.shape[1])

    def k2(g0_ref, g1_ref, h1p_ref, p_ref, b_ref, t2_ref):
        z1 = jnp.maximum(
            p_ref[...] * (g0_ref[:n, :] + g1_ref[:n, :] + h1p_ref[...])
            + b_ref[...], 0.0)
        t2_ref[...] = p_ref[...] * z1

    t2 = _tc(k2, jax.ShapeDtypeStruct((n, W1.shape[1]), jnp.float32),
             g1[0], g1[1], h1p, p, b1r)

    g2 = _sc_aggregate(t2, ei_rs, z64, n_pad, nchunks, W1.shape[1])

    def k3(g0_ref, g1_ref, t2_ref, p_ref, w2_ref, b2_ref, w3_ref, h3p_ref):
        a2 = p_ref[...] * (g0_ref[:n, :] + g1_ref[:n, :] + t2_ref[...])
        z2 = jnp.maximum(_dot(a2, w2_ref[...]) + b2_ref[...], 0.0)
        h3p_ref[...] = p_ref[...] * _dot(z2, w3_ref[...])

    h3p = _tc(k3, jax.ShapeDtypeStruct((n, W3.shape[1]), jnp.float32),
              g2[0], g2[1], t2, p, W2, b2r, W3)

    g3 = _sc_aggregate(h3p, ei_rs, z64, n_pad, nchunks, W3.shape[1])

    def k4(g0_ref, g1_ref, h3p_ref, p_ref, b3_ref, w4_ref, h4p_ref):
        z3 = jnp.maximum(
            p_ref[...] * (g0_ref[:n, :] + g1_ref[:n, :] + h3p_ref[...])
            + b3_ref[...], 0.0)
        h4p_ref[...] = p_ref[...] * _dot(z3, w4_ref[...])

    h4p = _tc(k4, jax.ShapeDtypeStruct((n, c4p), jnp.float32),
              g3[0], g3[1], h3p, p, b3r, W4p)

    g4 = _sc_aggregate(h4p, ei_rs, zc4, n_pad, nchunks, c4p)

    def k5(g0_ref, g1_ref, h4p_ref, p_ref, b4_ref, out_ref):
        z4 = p_ref[...] * (g0_ref[:n, :] + g1_ref[:n, :] + h4p_ref[...])
        z = z4[:, :c4] + b4_ref[...]
        m = jnp.max(z, axis=1, keepdims=True)
        zs = z - m
        lse = jnp.log(jnp.sum(jnp.exp(zs), axis=1, keepdims=True))
        out_ref[...] = zs - lse

    return _tc(k5, jax.ShapeDtypeStruct((n, c4), jnp.float32),
               g4[0], g4[1], h4p, p, b4r)


# speedup vs baseline: 31.5122x; 1.1293x over previous
import functools

import jax
import jax.numpy as jnp
from jax import lax
from jax.experimental import pallas as pl
from jax.experimental.pallas import tpu as pltpu
from jax.experimental.pallas import tpu_sc as plsc

NC = 2
NS = 16
NW = NC * NS
CHUNK = 128


def _sc_mesh():
    return plsc.VectorSubcoreMesh(core_axis_name="c", subcore_axis_name="s")


_SC_PARAMS = pltpu.CompilerParams(use_tc_tiling_on_sc=False)


def _sc_degree(ei_rs, ones_blk, zeros, n_pad, nchunks):
    rpt = n_pad // NS
    base = nchunks // NW
    extra = nchunks - base * NW

    @functools.partial(
        pl.kernel,
        out_type=jax.ShapeDtypeStruct((NC, n_pad, 16), jnp.float32),
        mesh=_sc_mesh(),
        compiler_params=_SC_PARAMS,
        scratch_types=[
            pltpu.VMEM((base + 1, CHUNK), jnp.int32),
            pltpu.VMEM((CHUNK, 16), jnp.float32),
            pltpu.VMEM_SHARED((n_pad, 16), jnp.float32),
        ],
    )
    def k(ei_hbm, ones_hbm, zeros_hbm, out_hbm, idx_v, ones_v, acc):
        c = lax.axis_index("c")
        s = lax.axis_index("s")
        w = c * NS + s
        r0 = s * rpt
        pltpu.sync_copy(zeros_hbm.at[pl.ds(r0, rpt)], acc.at[pl.ds(r0, rpt)])
        pltpu.sync_copy(ei_hbm.at[1, pl.ds(w * base, base)],
                        idx_v.at[pl.ds(0, base)])

        @pl.when(w < extra)
        def _():
            pltpu.sync_copy(ei_hbm.at[1, pl.ds(base * NW + w, 1)],
                            idx_v.at[pl.ds(base, 1)])

        pltpu.sync_copy(ones_hbm, ones_v)
        plsc.subcore_barrier()

        @pl.loop(0, base)
        def _(i):
            pltpu.sync_copy(ones_v, acc.at[idx_v.at[i]], add=True)

        @pl.when(w < extra)
        def _():
            pltpu.sync_copy(ones_v, acc.at[idx_v.at[base]], add=True)

        plsc.subcore_barrier()
        pltpu.sync_copy(acc.at[pl.ds(r0, rpt)], out_hbm.at[c, pl.ds(r0, rpt)])

    return k(ei_rs, ones_blk, zeros)


def _sc_aggregate(h, ei_rs, zeros, n_pad, nchunks, d):
    rpt = n_pad // NS
    n = h.shape[0]
    stg = n // NS // 8 * 8
    stg_last = n - stg * (NS - 1)
    base = nchunks // NW
    extra = nchunks - base * NW

    @functools.partial(
        pl.kernel,
        out_type=jax.ShapeDtypeStruct((NC, n_pad, d), jnp.float32),
        mesh=_sc_mesh(),
        compiler_params=_SC_PARAMS,
        scratch_types=[
            pltpu.VMEM((base + 8, CHUNK), jnp.int32),
            pltpu.VMEM((base + 1, CHUNK), jnp.int32),
            pltpu.VMEM((3, CHUNK, d), jnp.float32),
            pltpu.VMEM_SHARED((n, d), jnp.float32),
            pltpu.VMEM_SHARED((n_pad, d), jnp.float32),
            [pltpu.SemaphoreType.DMA] * 3,
            [pltpu.SemaphoreType.DMA] * 3,
        ],
    )
    def k(h_hbm, ei_hbm, zeros_hbm, out_hbm,
          src_v, dst_v, rows_v, table, acc, semg, sems):
        c = lax.axis_index("c")
        s = lax.axis_index("s")
        w = c * NS + s
        r0 = s * rpt
        @pl.when(s < NS - 1)
        def _():
            pltpu.sync_copy(h_hbm.at[pl.ds(s * stg, stg)],
                            table.at[pl.ds(s * stg, stg)])

        @pl.when(s == NS - 1)
        def _():
            pltpu.sync_copy(h_hbm.at[pl.ds((NS - 1) * stg, stg_last)],
                            table.at[pl.ds((NS - 1) * stg, stg_last)])

        pltpu.sync_copy(zeros_hbm.at[pl.ds(r0, rpt)], acc.at[pl.ds(r0, rpt)])
        pltpu.sync_copy(ei_hbm.at[0, pl.ds(w * base, base)],
                        src_v.at[pl.ds(0, base)])
        pltpu.sync_copy(ei_hbm.at[0, pl.ds(w * base, 8)],
                        src_v.at[pl.ds(base, 8)])
        pltpu.sync_copy(ei_hbm.at[1, pl.ds(w * base, base)],
                        dst_v.at[pl.ds(0, base)])

        @pl.when(w < extra)
        def _():
            pltpu.sync_copy(ei_hbm.at[0, pl.ds(base * NW + w, 1)],
                            src_v.at[pl.ds(base + 4, 1)])
            pltpu.sync_copy(ei_hbm.at[1, pl.ds(base * NW + w, 1)],
                            dst_v.at[pl.ds(base, 1)])

        plsc.subcore_barrier()

        @pl.when(w < extra)
        def _():
            pltpu.sync_copy(table.at[src_v.at[base + 4]], rows_v.at[0])
            pltpu.sync_copy(rows_v.at[0], acc.at[dst_v.at[base]], add=True)

        def fire_gather(i, b):
            pltpu.async_copy(table.at[src_v.at[i]], rows_v.at[b], semg[b])

        def wait_gather(i, b):
            pltpu.make_async_copy(table.at[src_v.at[i]], rows_v.at[b],
                                  semg[b]).wait()

        def fire_scatter(i, b):
            pltpu.async_copy(rows_v.at[b], acc.at[dst_v.at[i]], sems[b],
                             add=True)

        def wait_scatter(i, b):
            pltpu.make_async_copy(rows_v.at[b], acc.at[dst_v.at[i]],
                                  sems[b]).wait()

        fire_gather(0, 0)
        fire_gather(1, 1)
        wait_gather(0, 0)
        fire_scatter(0, 0)
        fire_gather(2, 2)

        nmain = (base - 1) // 3 * 3 + 1

        @pl.loop(0, (base - 1) // 3)
        def _(j):
            for k in range(3):
                i = 3 * j + 1 + k
                b = (1 + k) % 3
                wait_gather(i, b)
                fire_scatter(i, b)
                wait_scatter(i - 1, k)
                fire_gather(i + 2, k)

        for i in range(nmain, base):
            wait_gather(i, i % 3)
            fire_scatter(i, i % 3)
        for i in range(base, nmain + 2):
            wait_gather(i, i % 3)
        for i in range(max(0, base - 3), base):
            wait_scatter(i, i % 3)
        plsc.subcore_barrier()
        pltpu.sync_copy(acc.at[pl.ds(r0, rpt)], out_hbm.at[c, pl.ds(r0, rpt)])

    return k(h, ei_rs, zeros)


def _dot(a, b):
    return jnp.dot(a, b, preferred_element_type=jnp.float32,
                   precision=lax.Precision.DEFAULT)


def _tc(body, out_shape, *args):
    return pl.pallas_call(body, out_shape=out_shape)(*args)


def kernel(x, edge_index, W1, b1, W2, b2, W3, b3, W4, b4):
    n, d_in = x.shape
    e = edge_index.shape[1]
    c4 = W4.shape[1]
    c4p = ((c4 + 15) // 16) * 16

    n_pad = -(-n // (8 * NS)) * (8 * NS)
    assert e % CHUNK == 0
    nchunks = e // CHUNK

    ei_rs = edge_index.astype(jnp.int32).reshape(2, nchunks, CHUNK)

    ones_blk = jnp.ones((CHUNK, 16), jnp.float32)
    z16 = jnp.zeros((n_pad, 16), jnp.float32)
    z64 = jnp.zeros((n_pad, 64), jnp.float32)
    zc4 = jnp.zeros((n_pad, c4p), jnp.float32)

    b1r = b1.reshape(1, -1)
    b2r = b2.reshape(1, -1)
    b3r = b3.reshape(1, -1)
    b4r = b4.reshape(1, -1)
    W4p = jnp.pad(W4, ((0, 0), (0, c4p - c4)))

    dp = _sc_degree(ei_rs, ones_blk, z16, n_pad, nchunks)

    def k0(x_ref, w_ref, h1_ref):
        h1_ref[...] = _dot(x_ref[...], w_ref[...])

    h1 = _tc(k0, jax.ShapeDtypeStruct((n, W1.shape[1]), jnp.float32), x, W1)

    def k1(h1_ref, dp0_ref, dp1_ref, h1p_ref, p_ref):
        deg = dp0_ref[:, 0:1] + dp1_ref[:, 0:1] + 1.0
        p = lax.rsqrt(deg)[:n]
        h1p_ref[...] = p * h1_ref[...]
        p_ref[...] = p

    h1p, p = _tc(k1, (jax.ShapeDtypeStruct((n, W1.shape[1]), jnp.float32),
                      jax.ShapeDtypeStruct((n, 1), jnp.float32)),
                 h1, dp[0], dp[1])

    g1 = _sc_aggregate(h1p, ei_rs, z64, n_pad, nchunks, W1.shape[1])

    def k2(g0_ref, g1_ref, h1p_ref, p_ref, b_ref, t2_ref):
        z1 = jnp.maximum(
            p_ref[...] * (g0_ref[:n, :] + g1_ref[:n, :] + h1p_ref[...])
            + b_ref[...], 0.0)
        t2_ref[...] = p_ref[...] * z1

    t2 = _tc(k2, jax.ShapeDtypeStruct((n, W1.shape[1]), jnp.float32),
             g1[0], g1[1], h1p, p, b1r)

    g2 = _sc_aggregate(t2, ei_rs, z64, n_pad, nchunks, W1.shape[1])

    def k3(g0_ref, g1_ref, t2_ref, p_ref, w2_ref, b2_ref, w3_ref, h3p_ref):
        a2 = p_ref[...] * (g0_ref[:n, :] + g1_ref[:n, :] + t2_ref[...])
        z2 = jnp.maximum(_dot(a2, w2_ref[...]) + b2_ref[...], 0.0)
        h3p_ref[...] = p_ref[...] * _dot(z2, w3_ref[...])

    h3p = _tc(k3, jax.ShapeDtypeStruct((n, W3.shape[1]), jnp.float32),
              g2[0], g2[1], t2, p, W2, b2r, W3)

    g3 = _sc_aggregate(h3p, ei_rs, z64, n_pad, nchunks, W3.shape[1])

    def k4(g0_ref, g1_ref, h3p_ref, p_ref, b3_ref, w4_ref, h4p_ref):
        z3 = jnp.maximum(
            p_ref[...] * (g0_ref[:n, :] + g1_ref[:n, :] + h3p_ref[...])
            + b3_ref[...], 0.0)
        h4p_ref[...] = p_ref[...] * _dot(z3, w4_ref[...])

    h4p = _tc(k4, jax.ShapeDtypeStruct((n, c4p), jnp.float32),
              g3[0], g3[1], h3p, p, b3r, W4p)

    g4 = _sc_aggregate(h4p, ei_rs, zc4, n_pad, nchunks, c4p)

    def k5(g0_ref, g1_ref, h4p_ref, p_ref, b4_ref, out_ref):
        z4 = p_ref[...] * (g0_ref[:n, :] + g1_ref[:n, :] + h4p_ref[...])
        z = z4[:, :c4] + b4_ref[...]
        m = jnp.max(z, axis=1, keepdims=True)
        zs = z - m
        lse = jnp.log(jnp.sum(jnp.exp(zs), axis=1, keepdims=True))
        out_ref[...] = zs - lse

    return _tc(k5, jax.ShapeDtypeStruct((n, c4), jnp.float32),
               g4[0], g4[1], h4p, p, b4r)


# speedup vs baseline: 38.1905x; 1.2119x over previous
import functools

import jax
import jax.numpy as jnp
from jax import lax
from jax.experimental import pallas as pl
from jax.experimental.pallas import tpu as pltpu
from jax.experimental.pallas import tpu_sc as plsc

NC = 2
NS = 16
NW = NC * NS
CHUNK = 128


def _sc_mesh():
    return plsc.VectorSubcoreMesh(core_axis_name="c", subcore_axis_name="s")


_SC_PARAMS = pltpu.CompilerParams(use_tc_tiling_on_sc=False)


def _sc_degree(ei_rs, ones_blk, zeros, n_pad, nchunks):
    rpt = n_pad // NS
    base = nchunks // NW
    extra = nchunks - base * NW

    @functools.partial(
        pl.kernel,
        out_type=jax.ShapeDtypeStruct((NC, n_pad, 16), jnp.float32),
        mesh=_sc_mesh(),
        compiler_params=_SC_PARAMS,
        scratch_types=[
            pltpu.VMEM((base + 1, CHUNK), jnp.int32),
            pltpu.VMEM((CHUNK, 16), jnp.float32),
            pltpu.VMEM_SHARED((n_pad, 16), jnp.float32),
        ],
    )
    def k(ei_hbm, ones_hbm, zeros_hbm, out_hbm, idx_v, ones_v, acc):
        c = lax.axis_index("c")
        s = lax.axis_index("s")
        w = c * NS + s
        r0 = s * rpt
        pltpu.sync_copy(zeros_hbm.at[pl.ds(r0, rpt)], acc.at[pl.ds(r0, rpt)])
        pltpu.sync_copy(ei_hbm.at[1, pl.ds(w * base, base)],
                        idx_v.at[pl.ds(0, base)])

        @pl.when(w < extra)
        def _():
            pltpu.sync_copy(ei_hbm.at[1, pl.ds(base * NW + w, 1)],
                            idx_v.at[pl.ds(base, 1)])

        pltpu.sync_copy(ones_hbm, ones_v)
        plsc.subcore_barrier()

        @pl.loop(0, base)
        def _(i):
            pltpu.sync_copy(ones_v, acc.at[idx_v.at[i]], add=True)

        @pl.when(w < extra)
        def _():
            pltpu.sync_copy(ones_v, acc.at[idx_v.at[base]], add=True)

        plsc.subcore_barrier()
        pltpu.sync_copy(acc.at[pl.ds(r0, rpt)], out_hbm.at[c, pl.ds(r0, rpt)])

    return k(ei_rs, ones_blk, zeros)


def _sc_aggregate(h, ei_rs, zeros, n_pad, nchunks, d):
    rpt = n_pad // NS
    n = h.shape[0]
    stg = n // NS // 8 * 8
    stg_last = n - stg * (NS - 1)
    base = nchunks // NW
    extra = nchunks - base * NW

    @functools.partial(
        pl.kernel,
        out_type=jax.ShapeDtypeStruct((n_pad, 128), jnp.float32),
        mesh=_sc_mesh(),
        compiler_params=_SC_PARAMS,
        scratch_types=[
            pltpu.VMEM((base + 8, CHUNK), jnp.int32),
            pltpu.VMEM((base + 1, CHUNK), jnp.int32),
            pltpu.VMEM((3, CHUNK, d), jnp.float32),
            pltpu.VMEM_SHARED((n, d), jnp.float32),
            pltpu.VMEM_SHARED((n_pad, d), jnp.float32),
            [pltpu.SemaphoreType.DMA] * 3,
            [pltpu.SemaphoreType.DMA] * 3,
        ],
    )
    def k(h_hbm, ei_hbm, zeros_hbm, out_hbm,
          src_v, dst_v, rows_v, table, acc, semg, sems):
        c = lax.axis_index("c")
        s = lax.axis_index("s")
        w = c * NS + s
        r0 = s * rpt
        @pl.when(s < NS - 1)
        def _():
            pltpu.sync_copy(h_hbm.at[pl.ds(s * stg, stg), pl.ds(0, d)],
                            table.at[pl.ds(s * stg, stg)])

        @pl.when(s == NS - 1)
        def _():
            pltpu.sync_copy(
                h_hbm.at[pl.ds((NS - 1) * stg, stg_last), pl.ds(0, d)],
                table.at[pl.ds((NS - 1) * stg, stg_last)])

        pltpu.sync_copy(zeros_hbm.at[pl.ds(r0, rpt)], acc.at[pl.ds(r0, rpt)])
        pltpu.sync_copy(ei_hbm.at[0, pl.ds(w * base, base)],
                        src_v.at[pl.ds(0, base)])
        pltpu.sync_copy(ei_hbm.at[0, pl.ds(w * base, 8)],
                        src_v.at[pl.ds(base, 8)])
        pltpu.sync_copy(ei_hbm.at[1, pl.ds(w * base, base)],
                        dst_v.at[pl.ds(0, base)])

        @pl.when(w < extra)
        def _():
            pltpu.sync_copy(ei_hbm.at[0, pl.ds(base * NW + w, 1)],
                            src_v.at[pl.ds(base + 4, 1)])
            pltpu.sync_copy(ei_hbm.at[1, pl.ds(base * NW + w, 1)],
                            dst_v.at[pl.ds(base, 1)])

        plsc.subcore_barrier()

        @pl.when(w < extra)
        def _():
            pltpu.sync_copy(table.at[src_v.at[base + 4]], rows_v.at[0])
            pltpu.sync_copy(rows_v.at[0], acc.at[dst_v.at[base]], add=True)

        def fire_gather(i, b):
            pltpu.async_copy(table.at[src_v.at[i]], rows_v.at[b], semg[b])

        def wait_gather(i, b):
            pltpu.make_async_copy(table.at[src_v.at[i]], rows_v.at[b],
                                  semg[b]).wait()

        def fire_scatter(i, b):
            pltpu.async_copy(rows_v.at[b], acc.at[dst_v.at[i]], sems[b],
                             add=True)

        def wait_scatter(i, b):
            pltpu.make_async_copy(rows_v.at[b], acc.at[dst_v.at[i]],
                                  sems[b]).wait()

        fire_gather(0, 0)
        fire_gather(1, 1)
        wait_gather(0, 0)
        fire_scatter(0, 0)
        fire_gather(2, 2)

        nmain = (base - 1) // 3 * 3 + 1

        @pl.loop(0, (base - 1) // 3)
        def _(j):
            for k in range(3):
                i = 3 * j + 1 + k
                b = (1 + k) % 3
                wait_gather(i, b)
                fire_scatter(i, b)
                wait_scatter(i - 1, k)
                fire_gather(i + 2, k)

        for i in range(nmain, base):
            wait_gather(i, i % 3)
            fire_scatter(i, i % 3)
        for i in range(base, nmain + 2):
            wait_gather(i, i % 3)
        for i in range(max(0, base - 3), base):
            wait_scatter(i, i % 3)
        plsc.subcore_barrier()
        pltpu.sync_copy(acc.at[pl.ds(r0, rpt)],
                        out_hbm.at[pl.ds(r0, rpt), pl.ds(64 * c, d)])

    return k(h, ei_rs, zeros)


def _dot(a, b):
    return jnp.dot(a, b, preferred_element_type=jnp.float32,
                   precision=lax.Precision.DEFAULT)


def _tc(body, out_shape, *args):
    return pl.pallas_call(body, out_shape=out_shape)(*args)


def kernel(x, edge_index, W1, b1, W2, b2, W3, b3, W4, b4):
    n, d_in = x.shape
    e = edge_index.shape[1]
    c4 = W4.shape[1]
    c4p = ((c4 + 15) // 16) * 16

    n_pad = -(-n // (8 * NS)) * (8 * NS)
    assert e % CHUNK == 0
    nchunks = e // CHUNK

    ei_rs = edge_index.astype(jnp.int32).reshape(2, nchunks, CHUNK)

    ones_blk = jnp.ones((CHUNK, 16), jnp.float32)
    z16 = jnp.zeros((n_pad, 16), jnp.float32)
    z64 = jnp.zeros((n_pad, 64), jnp.float32)
    zc4 = jnp.zeros((n_pad, c4p), jnp.float32)

    b1r = b1.reshape(1, -1)
    b2r = b2.reshape(1, -1)
    b3r = b3.reshape(1, -1)
    b4r = b4.reshape(1, -1)
    W4p = jnp.pad(W4, ((0, 0), (0, c4p - c4)))

    dp = _sc_degree(ei_rs, ones_blk, z16, n_pad, nchunks)

    def k0(x_ref, w_ref, h1_ref):
        h1_ref[...] = _dot(x_ref[...], w_ref[...])

    h1 = _tc(k0, jax.ShapeDtypeStruct((n, W1.shape[1]), jnp.float32), x, W1)

    def dup(v):
        if v.shape[1] < 64:
            v = jnp.concatenate(
                [v, jnp.zeros((v.shape[0], 64 - v.shape[1]), v.dtype)], 1)
        return jnp.concatenate([v, v], axis=1)

    def k1(h1_ref, dp0_ref, dp1_ref, h1p_ref, p_ref):
        deg = dp0_ref[:, 0:1] + dp1_ref[:, 0:1] + 1.0
        p = lax.rsqrt(deg)[:n]
        h1p_ref[...] = dup(p * h1_ref[...])
        p_ref[...] = p

    h1p, p = _tc(k1, (jax.ShapeDtypeStruct((n, 128), jnp.float32),
                      jax.ShapeDtypeStruct((n, 1), jnp.float32)),
                 h1, dp[0], dp[1])

    g1 = _sc_aggregate(h1p, ei_rs, z64, n_pad, nchunks, 64)

    def k2(g_ref, h1p_ref, p_ref, b_ref, t2_ref):
        gsum = g_ref[:n, :64] + g_ref[:n, 64:] + h1p_ref[:, :64]
        z1 = jnp.maximum(p_ref[...] * gsum + b_ref[...], 0.0)
        t2_ref[...] = dup(p_ref[...] * z1)

    t2 = _tc(k2, jax.ShapeDtypeStruct((n, 128), jnp.float32),
             g1, h1p, p, b1r)

    g2 = _sc_aggregate(t2, ei_rs, z64, n_pad, nchunks, 64)

    def k3(g_ref, t2_ref, p_ref, w2_ref, b2_ref, w3_ref, h3p_ref):
        a2 = p_ref[...] * (g_ref[:n, :64] + g_ref[:n, 64:] + t2_ref[:, :64])
        z2 = jnp.maximum(_dot(a2, w2_ref[...]) + b2_ref[...], 0.0)
        h3p_ref[...] = dup(p_ref[...] * _dot(z2, w3_ref[...]))

    h3p = _tc(k3, jax.ShapeDtypeStruct((n, 128), jnp.float32),
              g2, t2, p, W2, b2r, W3)

    g3 = _sc_aggregate(h3p, ei_rs, z64, n_pad, nchunks, 64)

    def k4(g_ref, h3p_ref, p_ref, b3_ref, w4_ref, h4p_ref):
        z3 = jnp.maximum(
            p_ref[...] * (g_ref[:n, :64] + g_ref[:n, 64:] + h3p_ref[:, :64])
            + b3_ref[...], 0.0)
        h4p_ref[...] = dup(p_ref[...] * _dot(z3, w4_ref[...]))

    h4p = _tc(k4, jax.ShapeDtypeStruct((n, 128), jnp.float32),
              g3, h3p, p, b3r, W4p)

    g4 = _sc_aggregate(h4p, ei_rs, zc4, n_pad, nchunks, c4p)

    def k5(g_ref, h4p_ref, p_ref, b4_ref, out_ref):
        gsum = (g_ref[:n, :c4p] + g_ref[:n, 64:64 + c4p]
                + h4p_ref[:, :c4p])
        z4 = p_ref[...] * gsum
        z = z4[:, :c4] + b4_ref[...]
        m = jnp.max(z, axis=1, keepdims=True)
        zs = z - m
        lse = jnp.log(jnp.sum(jnp.exp(zs), axis=1, keepdims=True))
        out_ref[...] = zs - lse

    return _tc(k5, jax.ShapeDtypeStruct((n, c4), jnp.float32),
               g4, h4p, p, b4r)


# speedup vs baseline: 40.1007x; 1.0500x over previous
import functools

import jax
import jax.numpy as jnp
from jax import lax
from jax.experimental import pallas as pl
from jax.experimental.pallas import tpu as pltpu
from jax.experimental.pallas import tpu_sc as plsc

NC = 2
NS = 16
NW = NC * NS
CHUNK = 128


def _sc_mesh():
    return plsc.VectorSubcoreMesh(core_axis_name="c", subcore_axis_name="s")


_SC_PARAMS = pltpu.CompilerParams(use_tc_tiling_on_sc=False)


def _sc_degree(ei_rs, ones_blk, zeros, n_pad, nchunks):
    rpt = n_pad // NS
    base = nchunks // NW
    extra = nchunks - base * NW

    @functools.partial(
        pl.kernel,
        out_type=jax.ShapeDtypeStruct((NC, n_pad, 16), jnp.float32),
        mesh=_sc_mesh(),
        compiler_params=_SC_PARAMS,
        scratch_types=[
            pltpu.VMEM((base + 1, CHUNK), jnp.int32),
            pltpu.VMEM((CHUNK, 16), jnp.float32),
            pltpu.VMEM_SHARED((n_pad, 16), jnp.float32),
            [pltpu.SemaphoreType.DMA] * 4,
        ],
    )
    def k(ei_hbm, ones_hbm, zeros_hbm, out_hbm, idx_v, ones_v, acc, sems):
        c = lax.axis_index("c")
        s = lax.axis_index("s")
        w = c * NS + s
        r0 = s * rpt
        cz = pltpu.async_copy(zeros_hbm.at[pl.ds(r0, rpt)],
                              acc.at[pl.ds(r0, rpt)], sems[0])
        ci = pltpu.async_copy(ei_hbm.at[1, pl.ds(w * base, base)],
                              idx_v.at[pl.ds(0, base)], sems[1])
        co = pltpu.async_copy(ones_hbm, ones_v, sems[2])

        @pl.when(w < extra)
        def _():
            pltpu.sync_copy(ei_hbm.at[1, pl.ds(base * NW + w, 1)],
                            idx_v.at[pl.ds(base, 1)])

        cz.wait()
        ci.wait()
        co.wait()
        plsc.subcore_barrier()

        for b in range(min(4, base)):
            pltpu.async_copy(ones_v, acc.at[idx_v.at[b]], sems[b], add=True)

        @pl.loop(0, (base - 4) // 4)
        def _(j):
            for b in range(4):
                i = 4 * j + b
                pltpu.make_async_copy(ones_v, acc.at[idx_v.at[i]],
                                      sems[b]).wait()
                pltpu.async_copy(ones_v, acc.at[idx_v.at[i + 4]],
                                 sems[b], add=True)

        nmain = (base - 4) // 4 * 4
        for i in range(nmain + 4, base):
            pltpu.async_copy(ones_v, acc.at[idx_v.at[i]], sems[i % 4],
                             add=True)
        for i in range(nmain, base):
            pltpu.make_async_copy(ones_v, acc.at[idx_v.at[i]],
                                  sems[i % 4]).wait()

        @pl.when(w < extra)
        def _():
            pltpu.sync_copy(ones_v, acc.at[idx_v.at[base]], add=True)

        plsc.subcore_barrier()
        pltpu.sync_copy(acc.at[pl.ds(r0, rpt)], out_hbm.at[c, pl.ds(r0, rpt)])

    return k(ei_rs, ones_blk, zeros)


def _sc_aggregate(h, ei_rs, zeros, n_pad, nchunks, d):
    rpt = n_pad // NS
    n = h.shape[0]
    base = nchunks // NW
    extra = nchunks - base * NW

    @functools.partial(
        pl.kernel,
        out_type=jax.ShapeDtypeStruct((n_pad, 128), jnp.float32),
        mesh=_sc_mesh(),
        compiler_params=_SC_PARAMS,
        scratch_types=[
            pltpu.VMEM((base + 8, CHUNK), jnp.int32),
            pltpu.VMEM((base + 1, CHUNK), jnp.int32),
            pltpu.VMEM((3, CHUNK, d), jnp.float32),
            pltpu.VMEM_SHARED((n, d), jnp.float32),
            pltpu.VMEM_SHARED((n_pad, d), jnp.float32),
            [pltpu.SemaphoreType.DMA] * 3,
            [pltpu.SemaphoreType.DMA] * 3,
        ],
    )
    def k(h_hbm, ei_hbm, zeros_hbm, out_hbm,
          src_v, dst_v, rows_v, table, acc, semg, sems):
        c = lax.axis_index("c")
        s = lax.axis_index("s")
        w = c * NS + s
        r0 = s * rpt
        st0 = jnp.minimum(s * rpt, n - rpt)
        copies = []
        copies.append(pltpu.async_copy(
            h_hbm.at[pl.ds(st0, rpt), pl.ds(0, d)],
            table.at[pl.ds(st0, rpt)], semg[0]))
        copies.append(pltpu.async_copy(
            zeros_hbm.at[pl.ds(r0, rpt)], acc.at[pl.ds(r0, rpt)], semg[2]))
        copies.append(pltpu.async_copy(
            ei_hbm.at[0, pl.ds(w * base, base)],
            src_v.at[pl.ds(0, base)], sems[0]))
        copies.append(pltpu.async_copy(
            ei_hbm.at[0, pl.ds(w * base, 8)],
            src_v.at[pl.ds(base, 8)], sems[1]))
        copies.append(pltpu.async_copy(
            ei_hbm.at[1, pl.ds(w * base, base)],
            dst_v.at[pl.ds(0, base)], sems[2]))

        @pl.when(w < extra)
        def _():
            pltpu.sync_copy(ei_hbm.at[0, pl.ds(base * NW + w, 1)],
                            src_v.at[pl.ds(base + 4, 1)])
            pltpu.sync_copy(ei_hbm.at[1, pl.ds(base * NW + w, 1)],
                            dst_v.at[pl.ds(base, 1)])

        for cp in copies:
            cp.wait()
        plsc.subcore_barrier()

        @pl.when(w < extra)
        def _():
            pltpu.sync_copy(table.at[src_v.at[base + 4]], rows_v.at[0])
            pltpu.sync_copy(rows_v.at[0], acc.at[dst_v.at[base]], add=True)

        def fire_gather(i, b):
            pltpu.async_copy(table.at[src_v.at[i]], rows_v.at[b], semg[b])

        def wait_gather(i, b):
            pltpu.make_async_copy(table.at[src_v.at[i]], rows_v.at[b],
                                  semg[b]).wait()

        def fire_scatter(i, b):
            pltpu.async_copy(rows_v.at[b], acc.at[dst_v.at[i]], sems[b],
                             add=True)

        def wait_scatter(i, b):
            pltpu.make_async_copy(rows_v.at[b], acc.at[dst_v.at[i]],
                                  sems[b]).wait()

        fire_gather(0, 0)
        fire_gather(1, 1)
        wait_gather(0, 0)
        fire_scatter(0, 0)
        fire_gather(2, 2)

        nmain = (base - 1) // 3 * 3 + 1

        @pl.loop(0, (base - 1) // 3)
        def _(j):
            for k in range(3):
                i = 3 * j + 1 + k
                b = (1 + k) % 3
                wait_gather(i, b)
                fire_scatter(i, b)
                wait_scatter(i - 1, k)
                fire_gather(i + 2, k)

        for i in range(nmain, base):
            wait_gather(i, i % 3)
            fire_scatter(i, i % 3)
        for i in range(base, nmain + 2):
            wait_gather(i, i % 3)
        for i in range(max(0, base - 3), base):
            wait_scatter(i, i % 3)
        plsc.subcore_barrier()
        pltpu.sync_copy(acc.at[pl.ds(r0, rpt)],
                        out_hbm.at[pl.ds(r0, rpt), pl.ds(64 * c, d)])

    return k(h, ei_rs, zeros)


def _dot(a, b):
    return jnp.dot(a, b, preferred_element_type=jnp.float32,
                   precision=lax.Precision.DEFAULT)


def _tc(body, out_shape, *args):
    return pl.pallas_call(body, out_shape=out_shape)(*args)


def kernel(x, edge_index, W1, b1, W2, b2, W3, b3, W4, b4):
    n, d_in = x.shape
    e = edge_index.shape[1]
    c4 = W4.shape[1]
    c4p = ((c4 + 15) // 16) * 16

    n_pad = -(-n // (8 * NS)) * (8 * NS)
    assert e % CHUNK == 0
    nchunks = e // CHUNK

    ei_rs = edge_index.astype(jnp.int32).reshape(2, nchunks, CHUNK)

    ones_blk = jnp.ones((CHUNK, 16), jnp.float32)
    z16 = jnp.zeros((n_pad, 16), jnp.float32)
    z64 = jnp.zeros((n_pad, 64), jnp.float32)
    zc4 = jnp.zeros((n_pad, c4p), jnp.float32)

    b1r = b1.reshape(1, -1)
    b2r = b2.reshape(1, -1)
    b3r = b3.reshape(1, -1)
    b4r = b4.reshape(1, -1)
    W4p = jnp.pad(W4, ((0, 0), (0, c4p - c4)))

    dp = _sc_degree(ei_rs, ones_blk, z16, n_pad, nchunks)

    def k0(x_ref, w_ref, h1_ref):
        h1_ref[...] = _dot(x_ref[...], w_ref[...])

    h1 = _tc(k0, jax.ShapeDtypeStruct((n, W1.shape[1]), jnp.float32), x, W1)

    def dup(v):
        if v.shape[1] < 64:
            v = jnp.concatenate(
                [v, jnp.zeros((v.shape[0], 64 - v.shape[1]), v.dtype)], 1)
        return jnp.concatenate([v, v], axis=1)

    def k1(h1_ref, dp0_ref, dp1_ref, h1p_ref, p_ref):
        deg = dp0_ref[:, 0:1] + dp1_ref[:, 0:1] + 1.0
        p = lax.rsqrt(deg)[:n]
        h1p_ref[...] = dup(p * h1_ref[...])
        p_ref[...] = p

    h1p, p = _tc(k1, (jax.ShapeDtypeStruct((n, 128), jnp.float32),
                      jax.ShapeDtypeStruct((n, 1), jnp.float32)),
                 h1, dp[0], dp[1])

    g1 = _sc_aggregate(h1p, ei_rs, z64, n_pad, nchunks, 64)

    def k2(g_ref, h1p_ref, p_ref, b_ref, t2_ref):
        gsum = g_ref[:n, :64] + g_ref[:n, 64:] + h1p_ref[:, :64]
        z1 = jnp.maximum(p_ref[...] * gsum + b_ref[...], 0.0)
        t2_ref[...] = dup(p_ref[...] * z1)

    t2 = _tc(k2, jax.ShapeDtypeStruct((n, 128), jnp.float32),
             g1, h1p, p, b1r)

    g2 = _sc_aggregate(t2, ei_rs, z64, n_pad, nchunks, 64)

    def k3(g_ref, t2_ref, p_ref, w2_ref, b2_ref, w3_ref, h3p_ref):
        a2 = p_ref[...] * (g_ref[:n, :64] + g_ref[:n, 64:] + t2_ref[:, :64])
        z2 = jnp.maximum(_dot(a2, w2_ref[...]) + b2_ref[...], 0.0)
        h3p_ref[...] = dup(p_ref[...] * _dot(z2, w3_ref[...]))

    h3p = _tc(k3, jax.ShapeDtypeStruct((n, 128), jnp.float32),
              g2, t2, p, W2, b2r, W3)

    g3 = _sc_aggregate(h3p, ei_rs, z64, n_pad, nchunks, 64)

    def k4(g_ref, h3p_ref, p_ref, b3_ref, w4_ref, h4p_ref):
        z3 = jnp.maximum(
            p_ref[...] * (g_ref[:n, :64] + g_ref[:n, 64:] + h3p_ref[:, :64])
            + b3_ref[...], 0.0)
        h4p_ref[...] = dup(p_ref[...] * _dot(z3, w4_ref[...]))

    h4p = _tc(k4, jax.ShapeDtypeStruct((n, 128), jnp.float32),
              g3, h3p, p, b3r, W4p)

    g4 = _sc_aggregate(h4p, ei_rs, zc4, n_pad, nchunks, c4p)

    def k5(g_ref, h4p_ref, p_ref, b4_ref, out_ref):
        gsum = (g_ref[:n, :c4p] + g_ref[:n, 64:64 + c4p]
                + h4p_ref[:, :c4p])
        z4 = p_ref[...] * gsum
        z = z4[:, :c4] + b4_ref[...]
        m = jnp.max(z, axis=1, keepdims=True)
        zs = z - m
        lse = jnp.log(jnp.sum(jnp.exp(zs), axis=1, keepdims=True))
        out_ref[...] = zs - lse

    return _tc(k5, jax.ShapeDtypeStruct((n, c4), jnp.float32),
               g4, h4p, p, b4r)


# speedup vs baseline: 41.3506x; 1.0312x over previous
import functools

import jax
import jax.numpy as jnp
from jax import lax
from jax.experimental import pallas as pl
from jax.experimental.pallas import tpu as pltpu
from jax.experimental.pallas import tpu_sc as plsc

NC = 2
NS = 16
NW = NC * NS
CHUNK = 128


def _sc_mesh():
    return plsc.VectorSubcoreMesh(core_axis_name="c", subcore_axis_name="s")


_SC_PARAMS = pltpu.CompilerParams(use_tc_tiling_on_sc=False)


def _sc_degree(ei_rs, ones_blk, zeros, n_pad, nchunks):
    rpt = n_pad // NS
    base = nchunks // NW
    extra = nchunks - base * NW

    @functools.partial(
        pl.kernel,
        out_type=jax.ShapeDtypeStruct((n_pad, 128), jnp.float32),
        mesh=_sc_mesh(),
        compiler_params=_SC_PARAMS,
        scratch_types=[
            pltpu.VMEM((base + 1, CHUNK), jnp.int32),
            pltpu.VMEM((CHUNK, 16), jnp.float32),
            pltpu.VMEM_SHARED((n_pad, 16), jnp.float32),
            [pltpu.SemaphoreType.DMA] * 4,
        ],
    )
    def k(ei_hbm, ones_hbm, zeros_hbm, out_hbm, idx_v, ones_v, acc, sems):
        c = lax.axis_index("c")
        s = lax.axis_index("s")
        w = c * NS + s
        r0 = s * rpt
        cz = pltpu.async_copy(zeros_hbm.at[pl.ds(r0, rpt)],
                              acc.at[pl.ds(r0, rpt)], sems[0])
        ci = pltpu.async_copy(ei_hbm.at[1, pl.ds(w * base, base)],
                              idx_v.at[pl.ds(0, base)], sems[1])
        co = pltpu.async_copy(ones_hbm, ones_v, sems[2])

        @pl.when(w < extra)
        def _():
            pltpu.sync_copy(ei_hbm.at[1, pl.ds(base * NW + w, 1)],
                            idx_v.at[pl.ds(base, 1)])

        cz.wait()
        ci.wait()
        co.wait()
        plsc.subcore_barrier()

        for b in range(min(4, base)):
            pltpu.async_copy(ones_v, acc.at[idx_v.at[b]], sems[b], add=True)

        @pl.loop(0, (base - 4) // 4)
        def _(j):
            for b in range(4):
                i = 4 * j + b
                pltpu.make_async_copy(ones_v, acc.at[idx_v.at[i]],
                                      sems[b]).wait()
                pltpu.async_copy(ones_v, acc.at[idx_v.at[i + 4]],
                                 sems[b], add=True)

        nmain = (base - 4) // 4 * 4
        for i in range(nmain + 4, base):
            pltpu.async_copy(ones_v, acc.at[idx_v.at[i]], sems[i % 4],
                             add=True)
        for i in range(nmain, base):
            pltpu.make_async_copy(ones_v, acc.at[idx_v.at[i]],
                                  sems[i % 4]).wait()

        @pl.when(w < extra)
        def _():
            pltpu.sync_copy(ones_v, acc.at[idx_v.at[base]], add=True)

        plsc.subcore_barrier()
        pltpu.sync_copy(acc.at[pl.ds(r0, rpt)],
                        out_hbm.at[pl.ds(r0, rpt), pl.ds(16 * c, 16)])

    return k(ei_rs, ones_blk, zeros)


def _sc_aggregate(h, ei_rs, zeros, n_pad, nchunks, d):
    rpt = n_pad // NS
    n = h.shape[0]
    base = nchunks // NW
    extra = nchunks - base * NW

    @functools.partial(
        pl.kernel,
        out_type=jax.ShapeDtypeStruct((n_pad, 128), jnp.float32),
        mesh=_sc_mesh(),
        compiler_params=_SC_PARAMS,
        scratch_types=[
            pltpu.VMEM((base + 8, CHUNK), jnp.int32),
            pltpu.VMEM((base + 1, CHUNK), jnp.int32),
            pltpu.VMEM((3, CHUNK, d), jnp.float32),
            pltpu.VMEM_SHARED((n, d), jnp.float32),
            pltpu.VMEM_SHARED((n_pad, d), jnp.float32),
            [pltpu.SemaphoreType.DMA] * 3,
            [pltpu.SemaphoreType.DMA] * 3,
        ],
    )
    def k(h_hbm, ei_hbm, zeros_hbm, out_hbm,
          src_v, dst_v, rows_v, table, acc, semg, sems):
        c = lax.axis_index("c")
        s = lax.axis_index("s")
        w = c * NS + s
        r0 = s * rpt
        st0 = jnp.minimum(s * rpt, n - rpt)
        copies = []
        copies.append(pltpu.async_copy(
            h_hbm.at[pl.ds(st0, rpt), pl.ds(0, d)],
            table.at[pl.ds(st0, rpt)], semg[0]))
        copies.append(pltpu.async_copy(
            zeros_hbm.at[pl.ds(r0, rpt)], acc.at[pl.ds(r0, rpt)], semg[2]))
        copies.append(pltpu.async_copy(
            ei_hbm.at[0, pl.ds(w * base, base)],
            src_v.at[pl.ds(0, base)], sems[0]))
        copies.append(pltpu.async_copy(
            ei_hbm.at[0, pl.ds(w * base, 8)],
            src_v.at[pl.ds(base, 8)], sems[1]))
        copies.append(pltpu.async_copy(
            ei_hbm.at[1, pl.ds(w * base, base)],
            dst_v.at[pl.ds(0, base)], sems[2]))

        @pl.when(w < extra)
        def _():
            pltpu.sync_copy(ei_hbm.at[0, pl.ds(base * NW + w, 1)],
                            src_v.at[pl.ds(base + 4, 1)])
            pltpu.sync_copy(ei_hbm.at[1, pl.ds(base * NW + w, 1)],
                            dst_v.at[pl.ds(base, 1)])

        for cp in copies:
            cp.wait()
        plsc.subcore_barrier()

        @pl.when(w < extra)
        def _():
            pltpu.sync_copy(table.at[src_v.at[base + 4]], rows_v.at[0])
            pltpu.sync_copy(rows_v.at[0], acc.at[dst_v.at[base]], add=True)

        def fire_gather(i, b):
            pltpu.async_copy(table.at[src_v.at[i]], rows_v.at[b], semg[b])

        def wait_gather(i, b):
            pltpu.make_async_copy(table.at[src_v.at[i]], rows_v.at[b],
                                  semg[b]).wait()

        def fire_scatter(i, b):
            pltpu.async_copy(rows_v.at[b], acc.at[dst_v.at[i]], sems[b],
                             add=True)

        def wait_scatter(i, b):
            pltpu.make_async_copy(rows_v.at[b], acc.at[dst_v.at[i]],
                                  sems[b]).wait()

        fire_gather(0, 0)
        fire_gather(1, 1)
        wait_gather(0, 0)
        fire_scatter(0, 0)
        fire_gather(2, 2)

        nmain = (base - 1) // 3 * 3 + 1

        @pl.loop(0, (base - 1) // 3)
        def _(j):
            for k in range(3):
                i = 3 * j + 1 + k
                b = (1 + k) % 3
                wait_gather(i, b)
                fire_scatter(i, b)
                wait_scatter(i - 1, k)
                fire_gather(i + 2, k)

        for i in range(nmain, base):
            wait_gather(i, i % 3)
            fire_scatter(i, i % 3)
        for i in range(base, nmain + 2):
            wait_gather(i, i % 3)
        for i in range(max(0, base - 3), base):
            wait_scatter(i, i % 3)
        plsc.subcore_barrier()
        pltpu.sync_copy(acc.at[pl.ds(r0, rpt)],
                        out_hbm.at[pl.ds(r0, rpt), pl.ds(64 * c, d)])

    return k(h, ei_rs, zeros)


def _dot(a, b):
    return jnp.dot(a, b, preferred_element_type=jnp.float32,
                   precision=lax.Precision.DEFAULT)


def _tc(body, out_shape, *args):
    return pl.pallas_call(body, out_shape=out_shape)(*args)


def kernel(x, edge_index, W1, b1, W2, b2, W3, b3, W4, b4):
    n, d_in = x.shape
    e = edge_index.shape[1]
    c4 = W4.shape[1]
    c4p = ((c4 + 15) // 16) * 16

    n_pad = -(-n // (8 * NS)) * (8 * NS)
    assert e % CHUNK == 0
    nchunks = e // CHUNK

    ei_rs = edge_index.astype(jnp.int32).reshape(2, nchunks, CHUNK)

    ones_blk = jnp.ones((CHUNK, 16), jnp.float32)
    z16 = jnp.zeros((n_pad, 16), jnp.float32)
    z64 = jnp.zeros((n_pad, 64), jnp.float32)
    zc4 = jnp.zeros((n_pad, c4p), jnp.float32)

    b1r = b1.reshape(1, -1)
    b2r = b2.reshape(1, -1)
    b3r = b3.reshape(1, -1)
    b4r = b4.reshape(1, -1)
    W4p = jnp.pad(W4, ((0, 0), (0, c4p - c4)))

    dp = _sc_degree(ei_rs, ones_blk, z16, n_pad, nchunks)

    def k0(x_ref, w_ref, h1_ref):
        h1_ref[...] = _dot(x_ref[...], w_ref[...])

    h1 = _tc(k0, jax.ShapeDtypeStruct((n, W1.shape[1]), jnp.float32), x, W1)

    def dup(v):
        if v.shape[1] < 64:
            v = jnp.concatenate(
                [v, jnp.zeros((v.shape[0], 64 - v.shape[1]), v.dtype)], 1)
        return jnp.concatenate([v, v], axis=1)

    def k1(h1_ref, dp_ref, h1p_ref, p_ref):
        deg = dp_ref[:, 0:1] + dp_ref[:, 16:17] + 1.0
        p = lax.rsqrt(deg)[:n]
        h1p_ref[...] = dup(p * h1_ref[...])
        p_ref[...] = p

    h1p, p = _tc(k1, (jax.ShapeDtypeStruct((n, 128), jnp.float32),
                      jax.ShapeDtypeStruct((n, 1), jnp.float32)),
                 h1, dp)

    g1 = _sc_aggregate(h1p, ei_rs, z64, n_pad, nchunks, 64)

    def k2(g_ref, h1p_ref, p_ref, b_ref, t2_ref):
        gsum = g_ref[:n, :64] + g_ref[:n, 64:] + h1p_ref[:, :64]
        z1 = jnp.maximum(p_ref[...] * gsum + b_ref[...], 0.0)
        t2_ref[...] = dup(p_ref[...] * z1)

    t2 = _tc(k2, jax.ShapeDtypeStruct((n, 128), jnp.float32),
             g1, h1p, p, b1r)

    g2 = _sc_aggregate(t2, ei_rs, z64, n_pad, nchunks, 64)

    def k3(g_ref, t2_ref, p_ref, w2_ref, b2_ref, w3_ref, h3p_ref):
        a2 = p_ref[...] * (g_ref[:n, :64] + g_ref[:n, 64:] + t2_ref[:, :64])
        z2 = jnp.maximum(_dot(a2, w2_ref[...]) + b2_ref[...], 0.0)
        h3p_ref[...] = dup(p_ref[...] * _dot(z2, w3_ref[...]))

    h3p = _tc(k3, jax.ShapeDtypeStruct((n, 128), jnp.float32),
              g2, t2, p, W2, b2r, W3)

    g3 = _sc_aggregate(h3p, ei_rs, z64, n_pad, nchunks, 64)

    def k4(g_ref, h3p_ref, p_ref, b3_ref, w4_ref, h4p_ref):
        z3 = jnp.maximum(
            p_ref[...] * (g_ref[:n, :64] + g_ref[:n, 64:] + h3p_ref[:, :64])
            + b3_ref[...], 0.0)
        h4p_ref[...] = dup(p_ref[...] * _dot(z3, w4_ref[...]))

    h4p = _tc(k4, jax.ShapeDtypeStruct((n, 128), jnp.float32),
              g3, h3p, p, b3r, W4p)

    g4 = _sc_aggregate(h4p, ei_rs, zc4, n_pad, nchunks, c4p)

    def k5(g_ref, h4p_ref, p_ref, b4_ref, out_ref):
        gsum = (g_ref[:n, :c4p] + g_ref[:n, 64:64 + c4p]
                + h4p_ref[:, :c4p])
        z4 = p_ref[...] * gsum
        z = z4[:, :c4] + b4_ref[...]
        m = jnp.max(z, axis=1, keepdims=True)
        zs = z - m
        lse = jnp.log(jnp.sum(jnp.exp(zs), axis=1, keepdims=True))
        out_ref[...] = zs - lse

    return _tc(k5, jax.ShapeDtypeStruct((n, c4), jnp.float32),
               g4, h4p, p, b4r)
